# Initial kernel scaffold; baseline (speedup 1.0000x reference)
import jax
import jax.numpy as jnp
from jax.experimental import pallas as pl


def kernel(h, link_edge_index, direct_src, direct_dst, n_gym, fc1_W, fc1_b, fc2_W, fc2_b, sc1_W, sc1_b, bn1_w, bn1_b, fc3_W, fc3_b, fc4_W, fc4_b, sc2_W, sc2_b, bn2_w, bn2_b, fc5_W, fc5_b, fc6_W, fc6_b, sc3_W, sc3_b, bn3_w, bn3_b):
    raise NotImplementedError("write your pallas kernel here")


# speedup vs baseline: 19.4199x; 19.4199x over previous
import functools

import jax
import jax.numpy as jnp
from jax import lax
from jax.experimental import pallas as pl
from jax.experimental.pallas import tpu as pltpu
from jax.experimental.pallas import tpu_sc as plsc

NC, NS = 2, 16
NW = NC * NS
P = 8
LN = 128
N_ROAD = 100000
N_GYM = 1000
D = 6


def _build_agg(n_tab, n_acc, nch, g):
  Z = n_acc // NS
  mesh = plsc.VectorSubcoreMesh(
      core_axis_name="c", subcore_axis_name="s",
      num_cores=NC, num_subcores=NS)

  @functools.partial(
      pl.kernel,
      out_type=jax.ShapeDtypeStruct((NC, n_acc, P), jnp.float32),
      mesh=mesh,
      scratch_types=[
          pltpu.VMEM((g, LN), jnp.int32),
          pltpu.VMEM((g, LN), jnp.int32),
          pltpu.VMEM((g, LN, P), jnp.float32),
          pltpu.VMEM((Z, P), jnp.float32),
          pltpu.VMEM_SHARED((n_acc, P), jnp.float32),
          pltpu.SemaphoreType.DMA,
          pltpu.SemaphoreType.DMA,
      ],
      compiler_params=pltpu.CompilerParams(use_tc_tiling_on_sc=False),
  )
  def agg(tab_hbm, src_hbm, dst_hbm, zeros_hbm, out_hbm,
          src_v, dst_v, rows_v, stage_v, acc_sh, gsem, ssem):
    c = lax.axis_index("c")
    s = lax.axis_index("s")
    wid = s * NC + c
    pltpu.sync_copy(zeros_hbm, stage_v)
    pltpu.sync_copy(stage_v, acc_sh.at[pl.ds(s * Z, Z)])
    plsc.subcore_barrier()

    def body(i, carry):
      slab = wid * nch + i
      pltpu.sync_copy(src_hbm.at[slab], src_v)
      pltpu.sync_copy(dst_hbm.at[slab], dst_v)
      descs = [
          pltpu.async_copy(tab_hbm.at[src_v.at[k]], rows_v.at[k], gsem)
          for k in range(g)
      ]
      for d_ in descs:
        d_.wait()
      descs = [
          pltpu.async_copy(rows_v.at[k], acc_sh.at[dst_v.at[k]], ssem,
                           add=True)
          for k in range(g)
      ]
      for d_ in descs:
        d_.wait()
      return carry

    lax.fori_loop(0, nch, body, 0)
    plsc.subcore_barrier()
    pltpu.sync_copy(acc_sh.at[pl.ds(s * Z, Z)], stage_v)
    pltpu.sync_copy(stage_v, out_hbm.at[c, pl.ds(s * Z, Z)])

  return agg


_AGG_NCH, _AGG_G = 98, 8
_ACC_ROWS = 100352
_agg_main = _build_agg(N_ROAD, _ACC_ROWS, _AGG_NCH, _AGG_G)

_DIR_NCH, _DIR_G = 13, 1
_DACC_ROWS = 1024
_agg_dir = _build_agg(N_ROAD, _DACC_ROWS, _DIR_NCH, _DIR_G)

_B = 5000
_NB = N_ROAD // _B


def _dense_layer(h_pad, parts, W1, b1, bnw, bnb, W2, b2, Ws, bs):
  nf = 1.0 / N_ROAD
  hi = jax.lax.Precision.HIGHEST
  mm = functools.partial(jnp.matmul, precision=hi)

  def kern(h_ref, p_ref, W1_ref, b1_ref, bnw_ref, bnb_ref, W2_ref, b2_ref,
           Ws_ref, bs_ref, hn_ref, S_ref):
    p = pl.program_id(0)
    j = pl.program_id(1)
    sacc = p_ref[0] + p_ref[1]
    cnt = sacc[:, 6:7]
    m6 = jnp.where(cnt > 0, sacc[:, :6] / jnp.maximum(cnt, 1.0), 0.0)
    h6 = h_ref[:, :6]
    u = jnp.pad(h6, ((0, 0), (0, 6))) + jnp.pad(m6, ((0, 0), (6, 0)))

    @pl.when(jnp.logical_and(p == 0, j == 0))
    def _init():
      S_ref[...] = jnp.zeros_like(S_ref)

    @pl.when(p == 0)
    def _acc():
      uTu = lax.dot_general(u, u, (((0,), (0,)), ((), ())),
                            preferred_element_type=jnp.float32,
                            precision=hi)
      s1 = jnp.sum(u, axis=0, keepdims=True)
      S_ref[...] += (jnp.pad(uTu, ((0, 4), (0, 116))) +
                     jnp.pad(s1, ((12, 3), (0, 116))))

    @pl.when(p == 1)
    def _emit():
      W1m = W1_ref[...]
      z = mm(u, W1m) + b1_ref[...]
      S = S_ref[...]
      s2 = S[0:12, 0:12]
      s1 = S[12:13, 0:12]
      mu = s1 * nf
      muz = mm(mu, W1m) + b1_ref[...]
      outer = lax.dot_general(mu, mu, (((0,), (0,)), ((), ())),
                                preferred_element_type=jnp.float32,
                                precision=hi)
      C = s2 * nf - outer
      varz = jnp.sum(W1m * mm(C, W1m), axis=0, keepdims=True)
      inv = 1.0 / jnp.sqrt(varz + 1e-5)
      zb = (z - muz) * (inv * bnw_ref[...]) + bnb_ref[...]
      zl = jnp.where(zb >= 0, zb, 0.01 * zb)
      hn = (mm(zl, W2_ref[...]) + b2_ref[...] +
            mm(z, Ws_ref[...]) + bs_ref[...])
      hn8 = jnp.pad(hn, ((0, 0), (0, 2)))
      col6 = jnp.pad(jnp.ones((_B, 1), jnp.float32), ((0, 0), (6, 1)))
      hn_ref[...] = hn8 + col6

  full = lambda shape: pl.BlockSpec(shape, lambda p, j: tuple(0 for _ in shape))
  hn, _ = pl.pallas_call(
      kern,
      grid=(2, _NB),
      in_specs=[
          pl.BlockSpec((_B, P), lambda p, j: (j, 0)),
          pl.BlockSpec((NC, _B, P), lambda p, j: (0, j, 0)),
          full((12, 12)), full((1, 12)), full((1, 12)), full((1, 12)),
          full((12, 6)), full((1, 6)), full((12, 6)), full((1, 6)),
      ],
      out_specs=[
          pl.BlockSpec((_B, P), lambda p, j: (j, 0)),
          pl.BlockSpec((16, 128), lambda p, j: (0, 0)),
      ],
      out_shape=[
          jax.ShapeDtypeStruct((N_ROAD, P), jnp.float32),
          jax.ShapeDtypeStruct((16, 128), jnp.float32),
      ],
  )(h_pad, parts, W1, b1, bnw, bnb, W2, b2, Ws, bs)
  return hn


def _finalize(parts_d):
  def kern(p_ref, out_ref):
    sacc = p_ref[0] + p_ref[1]
    sacc = sacc[0:N_GYM, :]
    cnt = sacc[:, 6:7]
    out_ref[...] = jnp.where(cnt > 0, sacc[:, :6] / jnp.maximum(cnt, 1.0),
                             0.0)

  return pl.pallas_call(
      kern,
      out_shape=jax.ShapeDtypeStruct((N_GYM, D), jnp.float32),
  )(parts_d)


def _pad_edges(src, dst, trash, nch, g):
  e = src.shape[0]
  tot = NW * nch * g * LN
  src_p = jnp.concatenate(
      [src, jnp.zeros((tot - e,), jnp.int32)]).reshape(NW * nch, g, LN)
  dst_p = jnp.concatenate(
      [dst, jnp.full((tot - e,), trash, jnp.int32)]).reshape(NW * nch, g, LN)
  return src_p, dst_p


def kernel(h, link_edge_index, direct_src, direct_dst, n_gym,
           fc1_W, fc1_b, fc2_W, fc2_b, sc1_W, sc1_b, bn1_w, bn1_b,
           fc3_W, fc3_b, fc4_W, fc4_b, sc2_W, sc2_b, bn2_w, bn2_b,
           fc5_W, fc5_b, fc6_W, fc6_b, sc3_W, sc3_b, bn3_w, bn3_b):
  del n_gym
  ones = jnp.ones((N_ROAD, 1), jnp.float32)
  zer = jnp.zeros((N_ROAD, 1), jnp.float32)
  h0 = jnp.concatenate([h, ones, zer], axis=1)

  src_p, dst_p = _pad_edges(link_edge_index[0], link_edge_index[1],
                            N_ROAD, _AGG_NCH, _AGG_G)
  z_main = jnp.zeros((_ACC_ROWS // NS, P), jnp.float32)
  dsrc_p, ddst_p = _pad_edges(direct_src, direct_dst, N_GYM,
                              _DIR_NCH, _DIR_G)
  z_dir = jnp.zeros((_DACC_ROWS // NS, P), jnp.float32)

  r1 = lambda a: a.reshape(1, -1)

  parts = _agg_main(h0, src_p, dst_p, z_main)
  h1 = _dense_layer(h0, parts, fc1_W, r1(fc1_b), r1(bn1_w), r1(bn1_b),
                    fc2_W, r1(fc2_b), sc1_W, r1(sc1_b))
  parts = _agg_main(h1, src_p, dst_p, z_main)
  h2 = _dense_layer(h1, parts, fc3_W, r1(fc3_b), r1(bn2_w), r1(bn2_b),
                    fc4_W, r1(fc4_b), sc2_W, r1(sc2_b))
  parts = _agg_main(h2, src_p, dst_p, z_main)
  h3 = _dense_layer(h2, parts, fc5_W, r1(fc5_b), r1(bn3_w), r1(bn3_b),
                    fc6_W, r1(fc6_b), sc3_W, r1(sc3_b))

  parts_d = _agg_dir(h3, dsrc_p, ddst_p, z_dir)
  return _finalize(parts_d)


# speedup vs baseline: 41.6376x; 2.1441x over previous
import functools

import jax
import jax.numpy as jnp
from jax import lax
from jax.experimental import pallas as pl
from jax.experimental.pallas import tpu as pltpu
from jax.experimental.pallas import tpu_sc as plsc

NC, NS = 2, 16
NW = NC * NS
P = 8
LN = 128
N_ROAD = 100000
N_GYM = 1000
D = 6


def _build_agg(n_tab, n_acc, nch, g):
  Z = n_acc // NS
  mesh = plsc.VectorSubcoreMesh(
      core_axis_name="c", subcore_axis_name="s",
      num_cores=NC, num_subcores=NS)

  @functools.partial(
      pl.kernel,
      out_type=jax.ShapeDtypeStruct((NC, n_acc, P), jnp.float32),
      mesh=mesh,
      scratch_types=[
          pltpu.VMEM((g, LN), jnp.int32),
          pltpu.VMEM((g, LN), jnp.int32),
          pltpu.VMEM((g, LN, P), jnp.float32),
          pltpu.VMEM((Z, P), jnp.float32),
          pltpu.VMEM_SHARED((n_acc, P), jnp.float32),
          pltpu.SemaphoreType.DMA,
          pltpu.SemaphoreType.DMA,
      ],
      compiler_params=pltpu.CompilerParams(use_tc_tiling_on_sc=False),
  )
  def agg(tab_hbm, src_hbm, dst_hbm, zeros_hbm, out_hbm,
          src_v, dst_v, rows_v, stage_v, acc_sh, gsem, ssem):
    c = lax.axis_index("c")
    s = lax.axis_index("s")
    wid = s * NC + c
    pltpu.sync_copy(zeros_hbm, stage_v)
    pltpu.sync_copy(stage_v, acc_sh.at[pl.ds(s * Z, Z)])
    plsc.subcore_barrier()

    def body(i, carry):
      slab = wid * nch + i
      pltpu.sync_copy(src_hbm.at[slab], src_v)
      pltpu.sync_copy(dst_hbm.at[slab], dst_v)
      descs = [
          pltpu.async_copy(tab_hbm.at[src_v.at[k]], rows_v.at[k], gsem)
          for k in range(g)
      ]
      for d_ in descs:
        d_.wait()
      descs = [
          pltpu.async_copy(rows_v.at[k], acc_sh.at[dst_v.at[k]], ssem,
                           add=True)
          for k in range(g)
      ]
      for d_ in descs:
        d_.wait()
      return carry

    lax.fori_loop(0, nch, body, 0)
    plsc.subcore_barrier()
    pltpu.sync_copy(acc_sh.at[pl.ds(s * Z, Z)], stage_v)
    pltpu.sync_copy(stage_v, out_hbm.at[c, pl.ds(s * Z, Z)])

  return agg


_AGG_NCH, _AGG_G = 98, 8
_ACC_ROWS = 100352
_agg_main = _build_agg(N_ROAD, _ACC_ROWS, _AGG_NCH, _AGG_G)

_DIR_NCH, _DIR_G = 13, 1
_DACC_ROWS = 1024
_agg_dir = _build_agg(N_ROAD, _DACC_ROWS, _DIR_NCH, _DIR_G)

_NP = N_ROAD // 16
_PR = _ACC_ROWS // 16


def _bd(w8):
  return jnp.kron(jnp.eye(16, dtype=jnp.float32), w8)


def _tile16(v8):
  return jnp.tile(v8.reshape(1, 8), (1, 16))


def _dense_consts(W1, b1, bnw, bnb, W2, b2, Ws, bs):
  pad = jnp.pad
  mats = [
      _bd(pad(W1[:6, 0:8], ((0, 2), (0, 0)))),
      _bd(pad(W1[:6, 8:12], ((0, 2), (0, 4)))),
      _bd(pad(W1[6:12, 0:8], ((0, 2), (0, 0)))),
      _bd(pad(W1[6:12, 8:12], ((0, 2), (0, 4)))),
      _bd(pad(W2[0:8, :], ((0, 0), (0, 2)))),
      _bd(pad(W2[8:12, :], ((0, 4), (0, 2)))),
      _bd(pad(Ws[0:8, :], ((0, 0), (0, 2)))),
      _bd(pad(Ws[8:12, :], ((0, 4), (0, 2)))),
  ]
  hb = pad(b2 + bs, (0, 2)) + jnp.zeros((8,), jnp.float32).at[6].set(1.0)
  vecs = [
      _tile16(b1[0:8]), _tile16(pad(b1[8:12], (0, 4))),
      _tile16(bnw[0:8]), _tile16(pad(bnw[8:12], (0, 4))),
      _tile16(bnb[0:8]), _tile16(pad(bnb[8:12], (0, 4))),
      _tile16(hb),
  ]
  return mats, vecs


def _pick6():
  e = jnp.zeros((8, 8), jnp.float32).at[6, :].set(1.0)
  return _bd(e)


def _tilemat():
  return jnp.kron(jnp.ones((16, 16), jnp.float32), jnp.eye(8, dtype=jnp.float32))


def _dense_layer(h_p, parts_p, E_bd, T_mat, mats, vecs):
  hi = jax.lax.Precision.HIGHEST
  mm = functools.partial(jnp.matmul, precision=hi)
  nf = 1.0 / N_ROAD

  def kern(h_ref, p_ref, E_ref, T_ref,
           hA_ref, hB_ref, mA_ref, mB_ref,
           w2A_ref, w2B_ref, wsA_ref, wsB_ref,
           b1A_ref, b1B_ref, bnwA_ref, bnwB_ref, bnbA_ref, bnbB_ref, hb_ref,
           out_ref):
    pp = p_ref[0, 0:_NP, :] + p_ref[1, 0:_NP, :]
    cntb = mm(pp, E_ref[...])
    m_p = jnp.where(cntb > 0, pp / jnp.maximum(cntb, 1.0), 0.0)
    h_pk = h_ref[...]
    zA = mm(h_pk, hA_ref[...]) + mm(m_p, mA_ref[...]) + b1A_ref[...]
    zB = mm(h_pk, hB_ref[...]) + mm(m_p, mB_ref[...]) + b1B_ref[...]
    T = T_ref[...]
    muA = mm(jnp.sum(zA, axis=0, keepdims=True), T) * nf
    muB = mm(jnp.sum(zB, axis=0, keepdims=True), T) * nf
    qA = mm(jnp.sum(zA * zA, axis=0, keepdims=True), T) * nf
    qB = mm(jnp.sum(zB * zB, axis=0, keepdims=True), T) * nf
    invA = bnwA_ref[...] / jnp.sqrt(qA - muA * muA + 1e-5)
    invB = bnwB_ref[...] / jnp.sqrt(qB - muB * muB + 1e-5)
    zbA = (zA - muA) * invA + bnbA_ref[...]
    zbB = (zB - muB) * invB + bnbB_ref[...]
    zlA = jnp.where(zbA >= 0, zbA, 0.01 * zbA)
    zlB = jnp.where(zbB >= 0, zbB, 0.01 * zbB)
    out_ref[...] = (mm(zlA, w2A_ref[...]) + mm(zlB, w2B_ref[...]) +
                    mm(zA, wsA_ref[...]) + mm(zB, wsB_ref[...]) +
                    hb_ref[...])

  return pl.pallas_call(
      kern,
      out_shape=jax.ShapeDtypeStruct((_NP, 128), jnp.float32),
  )(h_p, parts_p, E_bd, T_mat, *mats, *vecs)


def _finalize(parts_d):
  def kern(p_ref, out_ref):
    sacc = p_ref[0] + p_ref[1]
    sacc = sacc[0:N_GYM, :]
    cnt = sacc[:, 6:7]
    out_ref[...] = jnp.where(cnt > 0, sacc[:, :6] / jnp.maximum(cnt, 1.0),
                             0.0)

  return pl.pallas_call(
      kern,
      out_shape=jax.ShapeDtypeStruct((N_GYM, D), jnp.float32),
  )(parts_d)


def _pad_edges(src, dst, trash, nch, g):
  e = src.shape[0]
  tot = NW * nch * g * LN
  src_p = jnp.concatenate(
      [src, jnp.zeros((tot - e,), jnp.int32)]).reshape(NW * nch, g, LN)
  dst_p = jnp.concatenate(
      [dst, jnp.full((tot - e,), trash, jnp.int32)]).reshape(NW * nch, g, LN)
  return src_p, dst_p


def kernel(h, link_edge_index, direct_src, direct_dst, n_gym,
           fc1_W, fc1_b, fc2_W, fc2_b, sc1_W, sc1_b, bn1_w, bn1_b,
           fc3_W, fc3_b, fc4_W, fc4_b, sc2_W, sc2_b, bn2_w, bn2_b,
           fc5_W, fc5_b, fc6_W, fc6_b, sc3_W, sc3_b, bn3_w, bn3_b):
  del n_gym
  ones = jnp.ones((N_ROAD, 1), jnp.float32)
  zer = jnp.zeros((N_ROAD, 1), jnp.float32)
  h0_p = jnp.concatenate([h, ones, zer], axis=1).reshape(_NP, 128)

  src_p, dst_p = _pad_edges(link_edge_index[0], link_edge_index[1],
                            N_ROAD, _AGG_NCH, _AGG_G)
  z_main = jnp.zeros((_ACC_ROWS // NS, P), jnp.float32)
  dsrc_p, ddst_p = _pad_edges(direct_src, direct_dst, N_GYM,
                              _DIR_NCH, _DIR_G)
  z_dir = jnp.zeros((_DACC_ROWS // NS, P), jnp.float32)

  E_bd = _pick6()
  T_mat = _tilemat()
  layer_consts = [
      _dense_consts(fc1_W, fc1_b, bn1_w, bn1_b, fc2_W, fc2_b, sc1_W, sc1_b),
      _dense_consts(fc3_W, fc3_b, bn2_w, bn2_b, fc4_W, fc4_b, sc2_W, sc2_b),
      _dense_consts(fc5_W, fc5_b, bn3_w, bn3_b, fc6_W, fc6_b, sc3_W, sc3_b),
  ]

  hp = h0_p
  for mats, vecs in layer_consts:
    tab = hp.reshape(N_ROAD, P)
    parts = _agg_main(tab, src_p, dst_p, z_main)
    parts_p = parts.reshape(NC, _PR, 128)
    hp = _dense_layer(hp, parts_p, E_bd, T_mat, mats, vecs)

  parts_d = _agg_dir(hp.reshape(N_ROAD, P), dsrc_p, ddst_p, z_dir)
  return _finalize(parts_d)


# speedup vs baseline: 56.8606x; 1.3656x over previous
import functools

import jax
import jax.numpy as jnp
from jax import lax
from jax.experimental import pallas as pl
from jax.experimental.pallas import tpu as pltpu
from jax.experimental.pallas import tpu_sc as plsc

NC, NS = 2, 16
NW = NC * NS
P = 8
LN = 128
N_ROAD = 100000
N_GYM = 1000
D = 6


def _build_agg(n_tab, n_acc, nch, g):
  Z = n_acc // NS
  mesh = plsc.VectorSubcoreMesh(
      core_axis_name="c", subcore_axis_name="s",
      num_cores=NC, num_subcores=NS)

  @functools.partial(
      pl.kernel,
      out_type=jax.ShapeDtypeStruct((NC, n_acc, P), jnp.float32),
      mesh=mesh,
      scratch_types=[
          pltpu.VMEM((2, g, LN), jnp.int32),
          pltpu.VMEM((2, g, LN), jnp.int32),
          pltpu.VMEM((2, g, LN, P), jnp.float32),
          pltpu.VMEM((Z, P), jnp.float32),
          pltpu.VMEM_SHARED((n_acc, P), jnp.float32),
          pltpu.SemaphoreType.DMA,
          pltpu.SemaphoreType.DMA,
          pltpu.SemaphoreType.DMA,
      ],
      compiler_params=pltpu.CompilerParams(use_tc_tiling_on_sc=False),
  )
  def agg(tab_hbm, src_hbm, dst_hbm, zeros_hbm, out_hbm,
          src_v, dst_v, rows_v, stage_v, acc_sh, isem, gsem, ssem):
    c = lax.axis_index("c")
    s = lax.axis_index("s")
    wid = s * NC + c
    base = wid * nch
    pltpu.sync_copy(zeros_hbm, stage_v)
    pltpu.sync_copy(stage_v, acc_sh.at[pl.ds(s * Z, Z)])
    pltpu.sync_copy(src_hbm.at[base], src_v.at[0])
    pltpu.sync_copy(dst_hbm.at[base], dst_v.at[0])
    plsc.subcore_barrier()
    for k in range(g):
      pltpu.async_copy(tab_hbm.at[src_v.at[0, k]], rows_v.at[0, k], gsem)

    def body(i, carry):
      par = lax.rem(i, 2)
      nxt = 1 - par
      nslab = base + i + 1
      pltpu.async_copy(src_hbm.at[nslab], src_v.at[nxt], isem)
      pltpu.async_copy(dst_hbm.at[nslab], dst_v.at[nxt], isem)
      for k in range(g):
        pltpu.make_async_copy(tab_hbm.at[src_v.at[par, k]],
                              rows_v.at[par, k], gsem).wait()
      pltpu.make_async_copy(src_hbm.at[nslab], src_v.at[nxt], isem).wait()
      pltpu.make_async_copy(dst_hbm.at[nslab], dst_v.at[nxt], isem).wait()
      for k in range(g):
        pltpu.async_copy(tab_hbm.at[src_v.at[nxt, k]], rows_v.at[nxt, k],
                         gsem)
      descs = [
          pltpu.async_copy(rows_v.at[par, k], acc_sh.at[dst_v.at[par, k]],
                           ssem, add=True)
          for k in range(g)
      ]
      for d_ in descs:
        d_.wait()
      return carry

    lax.fori_loop(0, nch, body, 0)
    last = nch % 2
    for k in range(g):
      pltpu.make_async_copy(tab_hbm.at[src_v.at[last, k]],
                            rows_v.at[last, k], gsem).wait()
    plsc.subcore_barrier()
    pltpu.sync_copy(acc_sh.at[pl.ds(s * Z, Z)], stage_v)
    pltpu.sync_copy(stage_v, out_hbm.at[c, pl.ds(s * Z, Z)])

  return agg


_AGG_NCH, _AGG_G = 98, 8
_ACC_ROWS = 100352
_agg_main = _build_agg(N_ROAD, _ACC_ROWS, _AGG_NCH, _AGG_G)

_DIR_NCH, _DIR_G = 13, 1
_DACC_ROWS = 1024
_agg_dir = _build_agg(N_ROAD, _DACC_ROWS, _DIR_NCH, _DIR_G)

_NP = N_ROAD // 16
_PR = _ACC_ROWS // 16


def _bd(w8):
  return jnp.kron(jnp.eye(16, dtype=jnp.float32), w8)


def _tile16(v8):
  return jnp.tile(v8.reshape(1, 8), (1, 16))


def _dense_consts(W1, b1, bnw, bnb, W2, b2, Ws, bs):
  pad = jnp.pad
  mats = [
      _bd(pad(W1[:6, 0:8], ((0, 2), (0, 0)))),
      _bd(pad(W1[:6, 8:12], ((0, 2), (0, 4)))),
      _bd(pad(W1[6:12, 0:8], ((0, 2), (0, 0)))),
      _bd(pad(W1[6:12, 8:12], ((0, 2), (0, 4)))),
      _bd(pad(W2[0:8, :], ((0, 0), (0, 2)))),
      _bd(pad(W2[8:12, :], ((0, 4), (0, 2)))),
      _bd(pad(Ws[0:8, :], ((0, 0), (0, 2)))),
      _bd(pad(Ws[8:12, :], ((0, 4), (0, 2)))),
  ]
  hb = pad(b2 + bs, (0, 2)) + jnp.zeros((8,), jnp.float32).at[6].set(1.0)
  vecs = [
      _tile16(b1[0:8]), _tile16(pad(b1[8:12], (0, 4))),
      _tile16(bnw[0:8]), _tile16(pad(bnw[8:12], (0, 4))),
      _tile16(bnb[0:8]), _tile16(pad(bnb[8:12], (0, 4))),
      _tile16(hb),
  ]
  return mats, vecs


def _pick6():
  e = jnp.zeros((8, 8), jnp.float32).at[6, :].set(1.0)
  return _bd(e)


def _tilemat():
  return jnp.kron(jnp.ones((16, 16), jnp.float32), jnp.eye(8, dtype=jnp.float32))


def _dense_layer(h_p, parts_p, E_bd, T_mat, mats, vecs):
  hi = jax.lax.Precision.HIGHEST
  mm = functools.partial(jnp.matmul, precision=hi)
  nf = 1.0 / N_ROAD

  def kern(h_ref, p_ref, E_ref, T_ref,
           hA_ref, hB_ref, mA_ref, mB_ref,
           w2A_ref, w2B_ref, wsA_ref, wsB_ref,
           b1A_ref, b1B_ref, bnwA_ref, bnwB_ref, bnbA_ref, bnbB_ref, hb_ref,
           out_ref):
    pp = p_ref[0, 0:_NP, :] + p_ref[1, 0:_NP, :]
    cntb = mm(pp, E_ref[...])
    m_p = jnp.where(cntb > 0, pp / jnp.maximum(cntb, 1.0), 0.0)
    h_pk = h_ref[...]
    zA = mm(h_pk, hA_ref[...]) + mm(m_p, mA_ref[...]) + b1A_ref[...]
    zB = mm(h_pk, hB_ref[...]) + mm(m_p, mB_ref[...]) + b1B_ref[...]
    T = T_ref[...]
    muA = mm(jnp.sum(zA, axis=0, keepdims=True), T) * nf
    muB = mm(jnp.sum(zB, axis=0, keepdims=True), T) * nf
    qA = mm(jnp.sum(zA * zA, axis=0, keepdims=True), T) * nf
    qB = mm(jnp.sum(zB * zB, axis=0, keepdims=True), T) * nf
    invA = bnwA_ref[...] / jnp.sqrt(qA - muA * muA + 1e-5)
    invB = bnwB_ref[...] / jnp.sqrt(qB - muB * muB + 1e-5)
    zbA = (zA - muA) * invA + bnbA_ref[...]
    zbB = (zB - muB) * invB + bnbB_ref[...]
    zlA = jnp.where(zbA >= 0, zbA, 0.01 * zbA)
    zlB = jnp.where(zbB >= 0, zbB, 0.01 * zbB)
    out_ref[...] = (mm(zlA, w2A_ref[...]) + mm(zlB, w2B_ref[...]) +
                    mm(zA, wsA_ref[...]) + mm(zB, wsB_ref[...]) +
                    hb_ref[...])

  return pl.pallas_call(
      kern,
      out_shape=jax.ShapeDtypeStruct((_NP, 128), jnp.float32),
  )(h_p, parts_p, E_bd, T_mat, *mats, *vecs)


def _finalize(parts_d):
  def kern(p_ref, out_ref):
    sacc = p_ref[0] + p_ref[1]
    sacc = sacc[0:N_GYM, :]
    cnt = sacc[:, 6:7]
    out_ref[...] = jnp.where(cnt > 0, sacc[:, :6] / jnp.maximum(cnt, 1.0),
                             0.0)

  return pl.pallas_call(
      kern,
      out_shape=jax.ShapeDtypeStruct((N_GYM, D), jnp.float32),
  )(parts_d)


def _pad_edges(src, dst, trash, nch, g):
  e = src.shape[0]
  tot = (NW * nch + 1) * g * LN
  src_p = jnp.concatenate(
      [src, jnp.zeros((tot - e,), jnp.int32)]).reshape(NW * nch + 1, g, LN)
  dst_p = jnp.concatenate(
      [dst, jnp.full((tot - e,), trash, jnp.int32)]).reshape(NW * nch + 1,
                                                             g, LN)
  return src_p, dst_p


def kernel(h, link_edge_index, direct_src, direct_dst, n_gym,
           fc1_W, fc1_b, fc2_W, fc2_b, sc1_W, sc1_b, bn1_w, bn1_b,
           fc3_W, fc3_b, fc4_W, fc4_b, sc2_W, sc2_b, bn2_w, bn2_b,
           fc5_W, fc5_b, fc6_W, fc6_b, sc3_W, sc3_b, bn3_w, bn3_b):
  del n_gym
  ones = jnp.ones((N_ROAD, 1), jnp.float32)
  zer = jnp.zeros((N_ROAD, 1), jnp.float32)
  h0_p = jnp.concatenate([h, ones, zer], axis=1).reshape(_NP, 128)

  src_p, dst_p = _pad_edges(link_edge_index[0], link_edge_index[1],
                            N_ROAD, _AGG_NCH, _AGG_G)
  z_main = jnp.zeros((_ACC_ROWS // NS, P), jnp.float32)
  dsrc_p, ddst_p = _pad_edges(direct_src, direct_dst, N_GYM,
                              _DIR_NCH, _DIR_G)
  z_dir = jnp.zeros((_DACC_ROWS // NS, P), jnp.float32)

  E_bd = _pick6()
  T_mat = _tilemat()
  layer_consts = [
      _dense_consts(fc1_W, fc1_b, bn1_w, bn1_b, fc2_W, fc2_b, sc1_W, sc1_b),
      _dense_consts(fc3_W, fc3_b, bn2_w, bn2_b, fc4_W, fc4_b, sc2_W, sc2_b),
      _dense_consts(fc5_W, fc5_b, bn3_w, bn3_b, fc6_W, fc6_b, sc3_W, sc3_b),
  ]

  hp = h0_p
  for mats, vecs in layer_consts:
    tab = hp.reshape(N_ROAD, P)
    parts = _agg_main(tab, src_p, dst_p, z_main)
    parts_p = parts.reshape(NC, _PR, 128)
    hp = _dense_layer(hp, parts_p, E_bd, T_mat, mats, vecs)

  parts_d = _agg_dir(hp.reshape(N_ROAD, P), dsrc_p, ddst_p, z_dir)
  return _finalize(parts_d)


# speedup vs baseline: 63.2240x; 1.1119x over previous
import functools

import jax
import jax.numpy as jnp
from jax import lax
from jax.experimental import pallas as pl
from jax.experimental.pallas import tpu as pltpu
from jax.experimental.pallas import tpu_sc as plsc

NC, NS = 2, 16
NW = NC * NS
P = 8
LN = 128
N_ROAD = 100000
N_GYM = 1000
D = 6
E_LINK = 3200000
_ACC_ROWS = 100352


def _build_agg(n_tab, n_acc, nch, g):
  Z = n_acc // NS
  mesh = plsc.VectorSubcoreMesh(
      core_axis_name="c", subcore_axis_name="s",
      num_cores=NC, num_subcores=NS)

  @functools.partial(
      pl.kernel,
      out_type=jax.ShapeDtypeStruct((NC, n_acc, P), jnp.float32),
      mesh=mesh,
      scratch_types=[
          pltpu.VMEM((2, g, LN), jnp.int32),
          pltpu.VMEM((2, g, LN), jnp.int32),
          pltpu.VMEM((2, g, LN, P), jnp.float32),
          pltpu.VMEM((Z, P), jnp.float32),
          pltpu.VMEM_SHARED((n_acc, P), jnp.float32),
          pltpu.SemaphoreType.DMA,
          pltpu.SemaphoreType.DMA,
          pltpu.SemaphoreType.DMA,
      ],
      compiler_params=pltpu.CompilerParams(use_tc_tiling_on_sc=False),
  )
  def agg(tab_hbm, src_hbm, dst_hbm, zeros_hbm, out_hbm,
          src_v, dst_v, rows_v, stage_v, acc_sh, isem, gsem, ssem):
    c = lax.axis_index("c")
    s = lax.axis_index("s")
    wid = s * NC + c
    base = wid * nch
    pltpu.sync_copy(zeros_hbm, stage_v)
    pltpu.sync_copy(stage_v, acc_sh.at[pl.ds(s * Z, Z)])
    pltpu.sync_copy(src_hbm.at[base], src_v.at[0])
    pltpu.sync_copy(dst_hbm.at[base], dst_v.at[0])
    plsc.subcore_barrier()
    for k in range(g):
      pltpu.async_copy(tab_hbm.at[src_v.at[0, k]], rows_v.at[0, k], gsem)

    def body(i, carry):
      par = lax.rem(i, 2)
      nxt = 1 - par
      nslab = base + i + 1
      pltpu.async_copy(src_hbm.at[nslab], src_v.at[nxt], isem)
      pltpu.async_copy(dst_hbm.at[nslab], dst_v.at[nxt], isem)
      for k in range(g):
        pltpu.make_async_copy(tab_hbm.at[src_v.at[par, k]],
                              rows_v.at[par, k], gsem).wait()
      pltpu.make_async_copy(src_hbm.at[nslab], src_v.at[nxt], isem).wait()
      pltpu.make_async_copy(dst_hbm.at[nslab], dst_v.at[nxt], isem).wait()
      for k in range(g):
        pltpu.async_copy(tab_hbm.at[src_v.at[nxt, k]], rows_v.at[nxt, k],
                         gsem)
      descs = [
          pltpu.async_copy(rows_v.at[par, k], acc_sh.at[dst_v.at[par, k]],
                           ssem, add=True)
          for k in range(g)
      ]
      for d_ in descs:
        d_.wait()
      return carry

    lax.fori_loop(0, nch, body, 0)
    last = nch % 2
    for k in range(g):
      pltpu.make_async_copy(tab_hbm.at[src_v.at[last, k]],
                            rows_v.at[last, k], gsem).wait()
    plsc.subcore_barrier()
    pltpu.sync_copy(acc_sh.at[pl.ds(s * Z, Z)], stage_v)
    pltpu.sync_copy(stage_v, out_hbm.at[c, pl.ds(s * Z, Z)])

  return agg


def _build_agg_flat(epw):
  n_acc = _ACC_ROWS
  Z = n_acc // NS
  g = _AGG_G
  chunk = g * LN
  nch = -(-epw // chunk)
  tail_off = epw - chunk
  dup = (nch - 1) * chunk - tail_off
  mesh = plsc.VectorSubcoreMesh(
      core_axis_name="c", subcore_axis_name="s",
      num_cores=NC, num_subcores=NS)

  @functools.partial(
      pl.kernel,
      out_type=jax.ShapeDtypeStruct((NC, n_acc, P), jnp.float32),
      mesh=mesh,
      scratch_types=[
          pltpu.VMEM((2, chunk), jnp.int32),
          pltpu.VMEM((2, chunk), jnp.int32),
          pltpu.VMEM((2, g, LN, P), jnp.float32),
          pltpu.VMEM((Z, P), jnp.float32),
          pltpu.VMEM_SHARED((n_acc, P), jnp.float32),
          pltpu.SemaphoreType.DMA,
          pltpu.SemaphoreType.DMA,
          pltpu.SemaphoreType.DMA,
      ],
      compiler_params=pltpu.CompilerParams(use_tc_tiling_on_sc=False),
  )
  def agg(tab_hbm, src_hbm, dst_hbm, zeros_hbm, out_hbm,
          src_v, dst_v, rows_v, stage_v, acc_sh, isem, gsem, ssem):
    c = lax.axis_index("c")
    s = lax.axis_index("s")
    wid = s * NC + c
    base = wid * epw
    pltpu.sync_copy(zeros_hbm, stage_v)
    pltpu.sync_copy(stage_v, acc_sh.at[pl.ds(s * Z, Z)])
    pltpu.sync_copy(src_hbm.at[pl.ds(base, chunk)], src_v.at[0])
    pltpu.sync_copy(dst_hbm.at[pl.ds(base, chunk)], dst_v.at[0])
    plsc.subcore_barrier()
    for k in range(g):
      pltpu.async_copy(tab_hbm.at[src_v.at[0, pl.ds(k * LN, LN)]],
                       rows_v.at[0, k], gsem)

    def body(i, carry):
      par = lax.rem(i, 2)
      nxt = 1 - par
      noff = base + lax.min((i + 1) * chunk, tail_off)
      pltpu.async_copy(src_hbm.at[pl.ds(noff, chunk)], src_v.at[nxt], isem)
      pltpu.async_copy(dst_hbm.at[pl.ds(noff, chunk)], dst_v.at[nxt], isem)
      for k in range(g):
        pltpu.make_async_copy(tab_hbm.at[src_v.at[par, pl.ds(k * LN, LN)]],
                              rows_v.at[par, k], gsem).wait()
      pltpu.make_async_copy(src_hbm.at[pl.ds(noff, chunk)], src_v.at[nxt],
                            isem).wait()
      pltpu.make_async_copy(dst_hbm.at[pl.ds(noff, chunk)], dst_v.at[nxt],
                            isem).wait()

      @pl.when(i == nch - 2)
      def _punch_tail():
        trash16 = jnp.full((16,), N_ROAD, jnp.int32)
        for j in range(dup // 16):
          dst_v[1, pl.ds(j * 16, 16)] = trash16

      for k in range(g):
        pltpu.async_copy(tab_hbm.at[src_v.at[nxt, pl.ds(k * LN, LN)]],
                         rows_v.at[nxt, k], gsem)
      descs = [
          pltpu.async_copy(rows_v.at[par, k],
                           acc_sh.at[dst_v.at[par, pl.ds(k * LN, LN)]],
                           ssem, add=True)
          for k in range(g)
      ]
      for d_ in descs:
        d_.wait()
      return carry

    lax.fori_loop(0, nch, body, 0)
    last = nch % 2
    for k in range(g):
      pltpu.make_async_copy(tab_hbm.at[src_v.at[last, pl.ds(k * LN, LN)]],
                            rows_v.at[last, k], gsem).wait()
    plsc.subcore_barrier()
    pltpu.sync_copy(acc_sh.at[pl.ds(s * Z, Z)], stage_v)
    pltpu.sync_copy(stage_v, out_hbm.at[c, pl.ds(s * Z, Z)])

  return agg


_AGG_G = 8

_agg_main = _build_agg_flat(E_LINK // NW)

_DIR_NCH, _DIR_G = 13, 1
_DACC_ROWS = 1024
_agg_dir = _build_agg(N_ROAD, _DACC_ROWS, _DIR_NCH, _DIR_G)

_NP = N_ROAD // 16
_PR = _ACC_ROWS // 16


def _bd(w8):
  return jnp.kron(jnp.eye(16, dtype=jnp.float32), w8)


def _tile16(v8):
  return jnp.tile(v8.reshape(1, 8), (1, 16))


def _dense_consts(W1, b1, bnw, bnb, W2, b2, Ws, bs):
  pad = jnp.pad
  mats = [
      _bd(pad(W1[:6, 0:8], ((0, 2), (0, 0)))),
      _bd(pad(W1[:6, 8:12], ((0, 2), (0, 4)))),
      _bd(pad(W1[6:12, 0:8], ((0, 2), (0, 0)))),
      _bd(pad(W1[6:12, 8:12], ((0, 2), (0, 4)))),
      _bd(pad(W2[0:8, :], ((0, 0), (0, 2)))),
      _bd(pad(W2[8:12, :], ((0, 4), (0, 2)))),
      _bd(pad(Ws[0:8, :], ((0, 0), (0, 2)))),
      _bd(pad(Ws[8:12, :], ((0, 4), (0, 2)))),
  ]
  hb = pad(b2 + bs, (0, 2)) + jnp.zeros((8,), jnp.float32).at[6].set(1.0)
  vecs = [
      _tile16(b1[0:8]), _tile16(pad(b1[8:12], (0, 4))),
      _tile16(bnw[0:8]), _tile16(pad(bnw[8:12], (0, 4))),
      _tile16(bnb[0:8]), _tile16(pad(bnb[8:12], (0, 4))),
      _tile16(hb),
  ]
  return mats, vecs


def _pick6():
  e = jnp.zeros((8, 8), jnp.float32).at[6, :].set(1.0)
  return _bd(e)


def _tilemat():
  return jnp.kron(jnp.ones((16, 16), jnp.float32), jnp.eye(8, dtype=jnp.float32))


def _dense_layer(h_p, parts_p, E_bd, T_mat, mats, vecs):
  hi = jax.lax.Precision.HIGHEST
  mm = functools.partial(jnp.matmul, precision=hi)
  nf = 1.0 / N_ROAD

  def kern(h_ref, p_ref, E_ref, T_ref,
           hA_ref, hB_ref, mA_ref, mB_ref,
           w2A_ref, w2B_ref, wsA_ref, wsB_ref,
           b1A_ref, b1B_ref, bnwA_ref, bnwB_ref, bnbA_ref, bnbB_ref, hb_ref,
           out_ref):
    pp = p_ref[0, 0:_NP, :] + p_ref[1, 0:_NP, :]
    cntb = mm(pp, E_ref[...])
    m_p = jnp.where(cntb > 0, pp / jnp.maximum(cntb, 1.0), 0.0)
    h_pk = h_ref[...]
    zA = mm(h_pk, hA_ref[...]) + mm(m_p, mA_ref[...]) + b1A_ref[...]
    zB = mm(h_pk, hB_ref[...]) + mm(m_p, mB_ref[...]) + b1B_ref[...]
    T = T_ref[...]
    muA = mm(jnp.sum(zA, axis=0, keepdims=True), T) * nf
    muB = mm(jnp.sum(zB, axis=0, keepdims=True), T) * nf
    qA = mm(jnp.sum(zA * zA, axis=0, keepdims=True), T) * nf
    qB = mm(jnp.sum(zB * zB, axis=0, keepdims=True), T) * nf
    invA = bnwA_ref[...] / jnp.sqrt(qA - muA * muA + 1e-5)
    invB = bnwB_ref[...] / jnp.sqrt(qB - muB * muB + 1e-5)
    zbA = (zA - muA) * invA + bnbA_ref[...]
    zbB = (zB - muB) * invB + bnbB_ref[...]
    zlA = jnp.where(zbA >= 0, zbA, 0.01 * zbA)
    zlB = jnp.where(zbB >= 0, zbB, 0.01 * zbB)
    out_ref[...] = (mm(zlA, w2A_ref[...]) + mm(zlB, w2B_ref[...]) +
                    mm(zA, wsA_ref[...]) + mm(zB, wsB_ref[...]) +
                    hb_ref[...])

  return pl.pallas_call(
      kern,
      out_shape=jax.ShapeDtypeStruct((_NP, 128), jnp.float32),
  )(h_p, parts_p, E_bd, T_mat, *mats, *vecs)


def _finalize(parts_d):
  def kern(p_ref, out_ref):
    sacc = p_ref[0] + p_ref[1]
    sacc = sacc[0:N_GYM, :]
    cnt = sacc[:, 6:7]
    out_ref[...] = jnp.where(cnt > 0, sacc[:, :6] / jnp.maximum(cnt, 1.0),
                             0.0)

  return pl.pallas_call(
      kern,
      out_shape=jax.ShapeDtypeStruct((N_GYM, D), jnp.float32),
  )(parts_d)


def _pad_edges(src, dst, trash, nch, g):
  e = src.shape[0]
  tot = (NW * nch + 1) * g * LN
  src_p = jnp.concatenate(
      [src, jnp.zeros((tot - e,), jnp.int32)]).reshape(NW * nch + 1, g, LN)
  dst_p = jnp.concatenate(
      [dst, jnp.full((tot - e,), trash, jnp.int32)]).reshape(NW * nch + 1,
                                                             g, LN)
  return src_p, dst_p


def kernel(h, link_edge_index, direct_src, direct_dst, n_gym,
           fc1_W, fc1_b, fc2_W, fc2_b, sc1_W, sc1_b, bn1_w, bn1_b,
           fc3_W, fc3_b, fc4_W, fc4_b, sc2_W, sc2_b, bn2_w, bn2_b,
           fc5_W, fc5_b, fc6_W, fc6_b, sc3_W, sc3_b, bn3_w, bn3_b):
  del n_gym
  col = jnp.zeros((1, P), jnp.float32).at[0, 6].set(1.0)
  h0_p = (jnp.pad(h, ((0, 0), (0, 2))) + col).reshape(_NP, 128)

  src_flat = link_edge_index[0]
  dst_flat = link_edge_index[1]
  z_main = jnp.zeros((_ACC_ROWS // NS, P), jnp.float32)
  dsrc_p, ddst_p = _pad_edges(direct_src, direct_dst, N_GYM,
                              _DIR_NCH, _DIR_G)
  z_dir = jnp.zeros((_DACC_ROWS // NS, P), jnp.float32)

  E_bd = _pick6()
  T_mat = _tilemat()
  layer_consts = [
      _dense_consts(fc1_W, fc1_b, bn1_w, bn1_b, fc2_W, fc2_b, sc1_W, sc1_b),
      _dense_consts(fc3_W, fc3_b, bn2_w, bn2_b, fc4_W, fc4_b, sc2_W, sc2_b),
      _dense_consts(fc5_W, fc5_b, bn3_w, bn3_b, fc6_W, fc6_b, sc3_W, sc3_b),
  ]

  hp = h0_p
  for mats, vecs in layer_consts:
    tab = hp.reshape(N_ROAD, P)
    parts = _agg_main(tab, src_flat, dst_flat, z_main)
    parts_p = parts.reshape(NC, _PR, 128)
    hp = _dense_layer(hp, parts_p, E_bd, T_mat, mats, vecs)

  parts_d = _agg_dir(hp.reshape(N_ROAD, P), dsrc_p, ddst_p, z_dir)
  return _finalize(parts_d)


# speedup vs baseline: 63.6918x; 1.0074x over previous
import functools

import jax
import jax.numpy as jnp
from jax import lax
from jax.experimental import pallas as pl
from jax.experimental.pallas import tpu as pltpu
from jax.experimental.pallas import tpu_sc as plsc

NC, NS = 2, 16
NW = NC * NS
P = 8
LN = 128
N_ROAD = 100000
N_GYM = 1000
D = 6
E_LINK = 3200000
_ACC_ROWS = 100352


def _build_agg(n_tab, n_acc, nch, g):
  Z = n_acc // NS
  mesh = plsc.VectorSubcoreMesh(
      core_axis_name="c", subcore_axis_name="s",
      num_cores=NC, num_subcores=NS)

  @functools.partial(
      pl.kernel,
      out_type=jax.ShapeDtypeStruct((NC, n_acc, P), jnp.float32),
      mesh=mesh,
      scratch_types=[
          pltpu.VMEM((2, g, LN), jnp.int32),
          pltpu.VMEM((2, g, LN), jnp.int32),
          pltpu.VMEM((2, g, LN, P), jnp.float32),
          pltpu.VMEM((Z, P), jnp.float32),
          pltpu.VMEM_SHARED((n_acc, P), jnp.float32),
          pltpu.SemaphoreType.DMA,
          pltpu.SemaphoreType.DMA,
          pltpu.SemaphoreType.DMA,
      ],
      compiler_params=pltpu.CompilerParams(use_tc_tiling_on_sc=False),
  )
  def agg(tab_hbm, src_hbm, dst_hbm, zeros_hbm, out_hbm,
          src_v, dst_v, rows_v, stage_v, acc_sh, isem, gsem, ssem):
    c = lax.axis_index("c")
    s = lax.axis_index("s")
    wid = s * NC + c
    base = wid * nch
    pltpu.sync_copy(zeros_hbm, stage_v)
    pltpu.sync_copy(stage_v, acc_sh.at[pl.ds(s * Z, Z)])
    pltpu.sync_copy(src_hbm.at[base], src_v.at[0])
    pltpu.sync_copy(dst_hbm.at[base], dst_v.at[0])
    plsc.subcore_barrier()
    for k in range(g):
      pltpu.async_copy(tab_hbm.at[src_v.at[0, k]], rows_v.at[0, k], gsem)

    def body(i, carry):
      par = lax.rem(i, 2)
      nxt = 1 - par
      nslab = base + i + 1
      pltpu.async_copy(src_hbm.at[nslab], src_v.at[nxt], isem)
      pltpu.async_copy(dst_hbm.at[nslab], dst_v.at[nxt], isem)
      for k in range(g):
        pltpu.make_async_copy(tab_hbm.at[src_v.at[par, k]],
                              rows_v.at[par, k], gsem).wait()
      pltpu.make_async_copy(src_hbm.at[nslab], src_v.at[nxt], isem).wait()
      pltpu.make_async_copy(dst_hbm.at[nslab], dst_v.at[nxt], isem).wait()
      for k in range(g):
        pltpu.async_copy(tab_hbm.at[src_v.at[nxt, k]], rows_v.at[nxt, k],
                         gsem)
      descs = [
          pltpu.async_copy(rows_v.at[par, k], acc_sh.at[dst_v.at[par, k]],
                           ssem, add=True)
          for k in range(g)
      ]
      for d_ in descs:
        d_.wait()
      return carry

    lax.fori_loop(0, nch, body, 0)
    last = nch % 2
    for k in range(g):
      pltpu.make_async_copy(tab_hbm.at[src_v.at[last, k]],
                            rows_v.at[last, k], gsem).wait()
    plsc.subcore_barrier()
    pltpu.sync_copy(acc_sh.at[pl.ds(s * Z, Z)], stage_v)
    pltpu.sync_copy(stage_v, out_hbm.at[c, pl.ds(s * Z, Z)])

  return agg


def _build_agg_flat(epw):
  n_acc = _ACC_ROWS
  Z = n_acc // NS
  g = _AGG_G
  chunk = g * LN
  nch = -(-epw // chunk)
  tail_off = epw - chunk
  dup = (nch - 1) * chunk - tail_off
  mesh = plsc.VectorSubcoreMesh(
      core_axis_name="c", subcore_axis_name="s",
      num_cores=NC, num_subcores=NS)

  @functools.partial(
      pl.kernel,
      out_type=jax.ShapeDtypeStruct((NC, n_acc, P), jnp.float32),
      mesh=mesh,
      scratch_types=[
          pltpu.VMEM((2, chunk), jnp.int32),
          pltpu.VMEM((2, chunk), jnp.int32),
          pltpu.VMEM((2, g, LN, P), jnp.float32),
          pltpu.VMEM((Z, P), jnp.float32),
          pltpu.VMEM_SHARED((n_acc, P), jnp.float32),
          pltpu.SemaphoreType.DMA,
          pltpu.SemaphoreType.DMA,
          pltpu.SemaphoreType.DMA,
      ],
      compiler_params=pltpu.CompilerParams(use_tc_tiling_on_sc=False),
  )
  def agg(tab_hbm, src_hbm, dst_hbm, zeros_hbm, out_hbm,
          src_v, dst_v, rows_v, stage_v, acc_sh, isem, gsem, ssem):
    c = lax.axis_index("c")
    s = lax.axis_index("s")
    wid = s * NC + c
    base = wid * epw
    pltpu.sync_copy(zeros_hbm, stage_v)
    pltpu.sync_copy(stage_v, acc_sh.at[pl.ds(s * Z, Z)])
    pltpu.sync_copy(src_hbm.at[pl.ds(base, chunk)], src_v.at[0])
    pltpu.sync_copy(dst_hbm.at[pl.ds(base, chunk)], dst_v.at[0])
    plsc.subcore_barrier()
    for k in range(g):
      pltpu.async_copy(tab_hbm.at[src_v.at[0, pl.ds(k * LN, LN)]],
                       rows_v.at[0, k], gsem)

    def body(i, carry):
      par = lax.rem(i, 2)
      nxt = 1 - par
      noff = base + lax.min((i + 1) * chunk, tail_off)
      pltpu.async_copy(src_hbm.at[pl.ds(noff, chunk)], src_v.at[nxt], isem)
      pltpu.async_copy(dst_hbm.at[pl.ds(noff, chunk)], dst_v.at[nxt], isem)
      for k in range(g):
        pltpu.make_async_copy(tab_hbm.at[src_v.at[par, pl.ds(k * LN, LN)]],
                              rows_v.at[par, k], gsem).wait()
      pltpu.make_async_copy(src_hbm.at[pl.ds(noff, chunk)], src_v.at[nxt],
                            isem).wait()
      pltpu.make_async_copy(dst_hbm.at[pl.ds(noff, chunk)], dst_v.at[nxt],
                            isem).wait()

      @pl.when(i == nch - 2)
      def _punch_tail():
        trash16 = jnp.full((16,), N_ROAD, jnp.int32)
        for j in range(dup // 16):
          dst_v[1, pl.ds(j * 16, 16)] = trash16

      for k in range(g):
        pltpu.async_copy(tab_hbm.at[src_v.at[nxt, pl.ds(k * LN, LN)]],
                         rows_v.at[nxt, k], gsem)
      descs = [
          pltpu.async_copy(rows_v.at[par, k],
                           acc_sh.at[dst_v.at[par, pl.ds(k * LN, LN)]],
                           ssem, add=True)
          for k in range(g)
      ]
      for d_ in descs:
        d_.wait()
      return carry

    lax.fori_loop(0, nch, body, 0)
    last = nch % 2
    for k in range(g):
      pltpu.make_async_copy(tab_hbm.at[src_v.at[last, pl.ds(k * LN, LN)]],
                            rows_v.at[last, k], gsem).wait()
    plsc.subcore_barrier()
    pltpu.sync_copy(acc_sh.at[pl.ds(s * Z, Z)], stage_v)
    pltpu.sync_copy(stage_v, out_hbm.at[c, pl.ds(s * Z, Z)])

  return agg


_AGG_G = 12

_agg_main = _build_agg_flat(E_LINK // NW)

_DIR_NCH, _DIR_G = 13, 1
_DACC_ROWS = 1024
_agg_dir = _build_agg(N_ROAD, _DACC_ROWS, _DIR_NCH, _DIR_G)

_NP = N_ROAD // 16
_PR = _ACC_ROWS // 16


def _bd(w8):
  return jnp.kron(jnp.eye(16, dtype=jnp.float32), w8)


def _tile16(v8):
  return jnp.tile(v8.reshape(1, 8), (1, 16))


def _dense_consts(W1, b1, bnw, bnb, W2, b2, Ws, bs):
  pad = jnp.pad
  mats = [
      _bd(pad(W1[:6, 0:8], ((0, 2), (0, 0)))),
      _bd(pad(W1[:6, 8:12], ((0, 2), (0, 4)))),
      _bd(pad(W1[6:12, 0:8], ((0, 2), (0, 0)))),
      _bd(pad(W1[6:12, 8:12], ((0, 2), (0, 4)))),
      _bd(pad(W2[0:8, :], ((0, 0), (0, 2)))),
      _bd(pad(W2[8:12, :], ((0, 4), (0, 2)))),
      _bd(pad(Ws[0:8, :], ((0, 0), (0, 2)))),
      _bd(pad(Ws[8:12, :], ((0, 4), (0, 2)))),
  ]
  hb = pad(b2 + bs, (0, 2)) + jnp.zeros((8,), jnp.float32).at[6].set(1.0)
  vecs = [
      _tile16(b1[0:8]), _tile16(pad(b1[8:12], (0, 4))),
      _tile16(bnw[0:8]), _tile16(pad(bnw[8:12], (0, 4))),
      _tile16(bnb[0:8]), _tile16(pad(bnb[8:12], (0, 4))),
      _tile16(hb),
  ]
  return mats, vecs


def _pick6():
  e = jnp.zeros((8, 8), jnp.float32).at[6, :].set(1.0)
  return _bd(e)


def _tilemat():
  return jnp.kron(jnp.ones((16, 16), jnp.float32), jnp.eye(8, dtype=jnp.float32))


def _dense_layer(h_p, parts_p, E_bd, T_mat, mats, vecs):
  hi = jax.lax.Precision.HIGHEST
  mm = functools.partial(jnp.matmul, precision=hi)
  nf = 1.0 / N_ROAD

  def kern(h_ref, p_ref, E_ref, T_ref,
           hA_ref, hB_ref, mA_ref, mB_ref,
           w2A_ref, w2B_ref, wsA_ref, wsB_ref,
           b1A_ref, b1B_ref, bnwA_ref, bnwB_ref, bnbA_ref, bnbB_ref, hb_ref,
           out_ref):
    pp = p_ref[0, 0:_NP, :] + p_ref[1, 0:_NP, :]
    cntb = mm(pp, E_ref[...])
    m_p = jnp.where(cntb > 0, pp / jnp.maximum(cntb, 1.0), 0.0)
    h_pk = h_ref[...]
    zA = mm(h_pk, hA_ref[...]) + mm(m_p, mA_ref[...]) + b1A_ref[...]
    zB = mm(h_pk, hB_ref[...]) + mm(m_p, mB_ref[...]) + b1B_ref[...]
    T = T_ref[...]
    muA = mm(jnp.sum(zA, axis=0, keepdims=True), T) * nf
    muB = mm(jnp.sum(zB, axis=0, keepdims=True), T) * nf
    qA = mm(jnp.sum(zA * zA, axis=0, keepdims=True), T) * nf
    qB = mm(jnp.sum(zB * zB, axis=0, keepdims=True), T) * nf
    invA = bnwA_ref[...] / jnp.sqrt(qA - muA * muA + 1e-5)
    invB = bnwB_ref[...] / jnp.sqrt(qB - muB * muB + 1e-5)
    zbA = (zA - muA) * invA + bnbA_ref[...]
    zbB = (zB - muB) * invB + bnbB_ref[...]
    zlA = jnp.where(zbA >= 0, zbA, 0.01 * zbA)
    zlB = jnp.where(zbB >= 0, zbB, 0.01 * zbB)
    out_ref[...] = (mm(zlA, w2A_ref[...]) + mm(zlB, w2B_ref[...]) +
                    mm(zA, wsA_ref[...]) + mm(zB, wsB_ref[...]) +
                    hb_ref[...])

  return pl.pallas_call(
      kern,
      out_shape=jax.ShapeDtypeStruct((_NP, 128), jnp.float32),
  )(h_p, parts_p, E_bd, T_mat, *mats, *vecs)


def _finalize(parts_d):
  def kern(p_ref, out_ref):
    sacc = p_ref[0] + p_ref[1]
    sacc = sacc[0:N_GYM, :]
    cnt = sacc[:, 6:7]
    out_ref[...] = jnp.where(cnt > 0, sacc[:, :6] / jnp.maximum(cnt, 1.0),
                             0.0)

  return pl.pallas_call(
      kern,
      out_shape=jax.ShapeDtypeStruct((N_GYM, D), jnp.float32),
  )(parts_d)


def _pad_edges(src, dst, trash, nch, g):
  e = src.shape[0]
  tot = (NW * nch + 1) * g * LN
  src_p = jnp.concatenate(
      [src, jnp.zeros((tot - e,), jnp.int32)]).reshape(NW * nch + 1, g, LN)
  dst_p = jnp.concatenate(
      [dst, jnp.full((tot - e,), trash, jnp.int32)]).reshape(NW * nch + 1,
                                                             g, LN)
  return src_p, dst_p


def kernel(h, link_edge_index, direct_src, direct_dst, n_gym,
           fc1_W, fc1_b, fc2_W, fc2_b, sc1_W, sc1_b, bn1_w, bn1_b,
           fc3_W, fc3_b, fc4_W, fc4_b, sc2_W, sc2_b, bn2_w, bn2_b,
           fc5_W, fc5_b, fc6_W, fc6_b, sc3_W, sc3_b, bn3_w, bn3_b):
  del n_gym
  col = jnp.zeros((1, P), jnp.float32).at[0, 6].set(1.0)
  h0_p = (jnp.pad(h, ((0, 0), (0, 2))) + col).reshape(_NP, 128)

  src_flat = link_edge_index[0]
  dst_flat = link_edge_index[1]
  z_main = jnp.zeros((_ACC_ROWS // NS, P), jnp.float32)
  dsrc_p, ddst_p = _pad_edges(direct_src, direct_dst, N_GYM,
                              _DIR_NCH, _DIR_G)
  z_dir = jnp.zeros((_DACC_ROWS // NS, P), jnp.float32)

  E_bd = _pick6()
  T_mat = _tilemat()
  layer_consts = [
      _dense_consts(fc1_W, fc1_b, bn1_w, bn1_b, fc2_W, fc2_b, sc1_W, sc1_b),
      _dense_consts(fc3_W, fc3_b, bn2_w, bn2_b, fc4_W, fc4_b, sc2_W, sc2_b),
      _dense_consts(fc5_W, fc5_b, bn3_w, bn3_b, fc6_W, fc6_b, sc3_W, sc3_b),
  ]

  hp = h0_p
  for mats, vecs in layer_consts:
    tab = hp.reshape(N_ROAD, P)
    parts = _agg_main(tab, src_flat, dst_flat, z_main)
    parts_p = parts.reshape(NC, _PR, 128)
    hp = _dense_layer(hp, parts_p, E_bd, T_mat, mats, vecs)

  parts_d = _agg_dir(hp.reshape(N_ROAD, P), dsrc_p, ddst_p, z_dir)
  return _finalize(parts_d)


# speedup vs baseline: 64.3645x; 1.0106x over previous
import functools

import jax
import jax.numpy as jnp
from jax import lax
from jax.experimental import pallas as pl
from jax.experimental.pallas import tpu as pltpu
from jax.experimental.pallas import tpu_sc as plsc

NC, NS = 2, 16
NW = NC * NS
P = 8
LN = 128
N_ROAD = 100000
N_GYM = 1000
D = 6
E_LINK = 3200000
_ACC_ROWS = 100352


def _build_agg(n_tab, n_acc, nch, g):
  Z = n_acc // NS
  mesh = plsc.VectorSubcoreMesh(
      core_axis_name="c", subcore_axis_name="s",
      num_cores=NC, num_subcores=NS)

  @functools.partial(
      pl.kernel,
      out_type=jax.ShapeDtypeStruct((NC, n_acc, P), jnp.float32),
      mesh=mesh,
      scratch_types=[
          pltpu.VMEM((2, g, LN), jnp.int32),
          pltpu.VMEM((2, g, LN), jnp.int32),
          pltpu.VMEM((2, g, LN, P), jnp.float32),
          pltpu.VMEM((Z, P), jnp.float32),
          pltpu.VMEM_SHARED((n_acc, P), jnp.float32),
          pltpu.SemaphoreType.DMA,
          pltpu.SemaphoreType.DMA,
          pltpu.SemaphoreType.DMA,
      ],
      compiler_params=pltpu.CompilerParams(use_tc_tiling_on_sc=False),
  )
  def agg(tab_hbm, src_hbm, dst_hbm, zeros_hbm, out_hbm,
          src_v, dst_v, rows_v, stage_v, acc_sh, isem, gsem, ssem):
    c = lax.axis_index("c")
    s = lax.axis_index("s")
    wid = s * NC + c
    base = wid * nch
    pltpu.sync_copy(zeros_hbm, stage_v)
    pltpu.sync_copy(stage_v, acc_sh.at[pl.ds(s * Z, Z)])
    pltpu.sync_copy(src_hbm.at[base], src_v.at[0])
    pltpu.sync_copy(dst_hbm.at[base], dst_v.at[0])
    plsc.subcore_barrier()
    for k in range(g):
      pltpu.async_copy(tab_hbm.at[src_v.at[0, k]], rows_v.at[0, k], gsem)

    def body(i, carry):
      par = lax.rem(i, 2)
      nxt = 1 - par
      nslab = base + i + 1
      pltpu.async_copy(src_hbm.at[nslab], src_v.at[nxt], isem)
      pltpu.async_copy(dst_hbm.at[nslab], dst_v.at[nxt], isem)
      for k in range(g):
        pltpu.make_async_copy(tab_hbm.at[src_v.at[par, k]],
                              rows_v.at[par, k], gsem).wait()
      pltpu.make_async_copy(src_hbm.at[nslab], src_v.at[nxt], isem).wait()
      pltpu.make_async_copy(dst_hbm.at[nslab], dst_v.at[nxt], isem).wait()
      for k in range(g):
        pltpu.async_copy(tab_hbm.at[src_v.at[nxt, k]], rows_v.at[nxt, k],
                         gsem)
      descs = [
          pltpu.async_copy(rows_v.at[par, k], acc_sh.at[dst_v.at[par, k]],
                           ssem, add=True)
          for k in range(g)
      ]
      for d_ in descs:
        d_.wait()
      return carry

    lax.fori_loop(0, nch, body, 0)
    last = nch % 2
    for k in range(g):
      pltpu.make_async_copy(tab_hbm.at[src_v.at[last, k]],
                            rows_v.at[last, k], gsem).wait()
    plsc.subcore_barrier()
    pltpu.sync_copy(acc_sh.at[pl.ds(s * Z, Z)], stage_v)
    pltpu.sync_copy(stage_v, out_hbm.at[c, pl.ds(s * Z, Z)])

  return agg


def _build_agg_flat(epw):
  n_acc = _ACC_ROWS
  Z = n_acc // NS
  g = _AGG_G
  chunk = g * LN
  nch = -(-epw // chunk)
  tail_off = epw - chunk
  dup = (nch - 1) * chunk - tail_off
  mesh = plsc.VectorSubcoreMesh(
      core_axis_name="c", subcore_axis_name="s",
      num_cores=NC, num_subcores=NS)

  @functools.partial(
      pl.kernel,
      out_type=jax.ShapeDtypeStruct((NC, n_acc, P), jnp.float32),
      mesh=mesh,
      scratch_types=[
          pltpu.VMEM((2, chunk), jnp.int32),
          pltpu.VMEM((2, chunk), jnp.int32),
          pltpu.VMEM((2, g, LN, P), jnp.float32),
          pltpu.VMEM((Z, P), jnp.float32),
          pltpu.VMEM_SHARED((n_acc, P), jnp.float32),
          pltpu.SemaphoreType.DMA,
          pltpu.SemaphoreType.DMA,
          pltpu.SemaphoreType.DMA,
      ],
      compiler_params=pltpu.CompilerParams(use_tc_tiling_on_sc=False),
  )
  def agg(tab_hbm, link_hbm, zeros_hbm, out_hbm,
          src_v, dst_v, rows_v, stage_v, acc_sh, isem, gsem, ssem):
    c = lax.axis_index("c")
    s = lax.axis_index("s")
    wid = s * NC + c
    base = wid * epw
    pltpu.sync_copy(zeros_hbm, stage_v)
    pltpu.sync_copy(stage_v, acc_sh.at[pl.ds(s * Z, Z)])
    pltpu.sync_copy(link_hbm.at[0, pl.ds(base, chunk)], src_v.at[0])
    pltpu.sync_copy(link_hbm.at[1, pl.ds(base, chunk)], dst_v.at[0])
    plsc.subcore_barrier()
    for k in range(g):
      pltpu.async_copy(tab_hbm.at[src_v.at[0, pl.ds(k * LN, LN)]],
                       rows_v.at[0, k], gsem)

    def body(i, carry):
      par = lax.rem(i, 2)
      nxt = 1 - par
      noff = base + lax.min((i + 1) * chunk, tail_off)
      pltpu.async_copy(link_hbm.at[0, pl.ds(noff, chunk)], src_v.at[nxt],
                       isem)
      pltpu.async_copy(link_hbm.at[1, pl.ds(noff, chunk)], dst_v.at[nxt],
                       isem)
      for k in range(g):
        pltpu.make_async_copy(tab_hbm.at[src_v.at[par, pl.ds(k * LN, LN)]],
                              rows_v.at[par, k], gsem).wait()
      pltpu.make_async_copy(link_hbm.at[0, pl.ds(noff, chunk)],
                            src_v.at[nxt], isem).wait()
      pltpu.make_async_copy(link_hbm.at[1, pl.ds(noff, chunk)],
                            dst_v.at[nxt], isem).wait()

      @pl.when(i == nch - 2)
      def _punch_tail():
        trash16 = jnp.full((16,), N_ROAD, jnp.int32)
        for j in range(dup // 16):
          dst_v[1, pl.ds(j * 16, 16)] = trash16

      for k in range(g):
        pltpu.async_copy(tab_hbm.at[src_v.at[nxt, pl.ds(k * LN, LN)]],
                         rows_v.at[nxt, k], gsem)
      descs = [
          pltpu.async_copy(rows_v.at[par, k],
                           acc_sh.at[dst_v.at[par, pl.ds(k * LN, LN)]],
                           ssem, add=True)
          for k in range(g)
      ]
      for d_ in descs:
        d_.wait()
      return carry

    lax.fori_loop(0, nch, body, 0)
    last = nch % 2
    for k in range(g):
      pltpu.make_async_copy(tab_hbm.at[src_v.at[last, pl.ds(k * LN, LN)]],
                            rows_v.at[last, k], gsem).wait()
    plsc.subcore_barrier()
    pltpu.sync_copy(acc_sh.at[pl.ds(s * Z, Z)], stage_v)
    pltpu.sync_copy(stage_v, out_hbm.at[c, pl.ds(s * Z, Z)])

  return agg


_AGG_G = 12

_agg_main = _build_agg_flat(E_LINK // NW)

_DIR_NCH, _DIR_G = 13, 1
_DACC_ROWS = 1024
_agg_dir = _build_agg(N_ROAD, _DACC_ROWS, _DIR_NCH, _DIR_G)

_NP = N_ROAD // 16
_PR = _ACC_ROWS // 16


def _bd(w8):
  return jnp.kron(jnp.eye(16, dtype=jnp.float32), w8)


def _tile16(v8):
  return jnp.tile(v8.reshape(1, 8), (1, 16))


def _dense_consts(W1, b1, bnw, bnb, W2, b2, Ws, bs):
  pad = jnp.pad
  w8s = jnp.stack([
      pad(W1[:6, 0:8], ((0, 2), (0, 0))),
      pad(W1[:6, 8:12], ((0, 2), (0, 4))),
      pad(W1[6:12, 0:8], ((0, 2), (0, 0))),
      pad(W1[6:12, 8:12], ((0, 2), (0, 4))),
      pad(W2[0:8, :], ((0, 0), (0, 2))),
      pad(W2[8:12, :], ((0, 4), (0, 2))),
      pad(Ws[0:8, :], ((0, 0), (0, 2))),
      pad(Ws[8:12, :], ((0, 4), (0, 2))),
  ])
  eye16 = jnp.eye(16, dtype=jnp.float32)
  mats = list(jnp.einsum('ij,nkl->nikjl', eye16, w8s).reshape(8, 128, 128))
  hb = pad(b2 + bs, (0, 2)) + jnp.zeros((8,), jnp.float32).at[6].set(1.0)
  vecs = [
      _tile16(b1[0:8]), _tile16(pad(b1[8:12], (0, 4))),
      _tile16(bnw[0:8]), _tile16(pad(bnw[8:12], (0, 4))),
      _tile16(bnb[0:8]), _tile16(pad(bnb[8:12], (0, 4))),
      _tile16(hb),
  ]
  return mats, vecs


def _pick6():
  e = jnp.zeros((8, 8), jnp.float32).at[6, :].set(1.0)
  return _bd(e)


def _tilemat():
  return jnp.kron(jnp.ones((16, 16), jnp.float32), jnp.eye(8, dtype=jnp.float32))


def _dense_layer(h_p, parts_p, E_bd, T_mat, mats, vecs):
  hi = jax.lax.Precision.HIGHEST
  mm = functools.partial(jnp.matmul, precision=hi)
  nf = 1.0 / N_ROAD

  def kern(h_ref, p_ref, E_ref, T_ref,
           hA_ref, hB_ref, mA_ref, mB_ref,
           w2A_ref, w2B_ref, wsA_ref, wsB_ref,
           b1A_ref, b1B_ref, bnwA_ref, bnwB_ref, bnbA_ref, bnbB_ref, hb_ref,
           out_ref):
    pp = p_ref[0, 0:_NP, :] + p_ref[1, 0:_NP, :]
    cntb = mm(pp, E_ref[...])
    m_p = jnp.where(cntb > 0, pp / jnp.maximum(cntb, 1.0), 0.0)
    h_pk = h_ref[...]
    zA = mm(h_pk, hA_ref[...]) + mm(m_p, mA_ref[...]) + b1A_ref[...]
    zB = mm(h_pk, hB_ref[...]) + mm(m_p, mB_ref[...]) + b1B_ref[...]
    T = T_ref[...]
    muA = mm(jnp.sum(zA, axis=0, keepdims=True), T) * nf
    muB = mm(jnp.sum(zB, axis=0, keepdims=True), T) * nf
    qA = mm(jnp.sum(zA * zA, axis=0, keepdims=True), T) * nf
    qB = mm(jnp.sum(zB * zB, axis=0, keepdims=True), T) * nf
    invA = bnwA_ref[...] / jnp.sqrt(qA - muA * muA + 1e-5)
    invB = bnwB_ref[...] / jnp.sqrt(qB - muB * muB + 1e-5)
    zbA = (zA - muA) * invA + bnbA_ref[...]
    zbB = (zB - muB) * invB + bnbB_ref[...]
    zlA = jnp.where(zbA >= 0, zbA, 0.01 * zbA)
    zlB = jnp.where(zbB >= 0, zbB, 0.01 * zbB)
    out_ref[...] = (mm(zlA, w2A_ref[...]) + mm(zlB, w2B_ref[...]) +
                    mm(zA, wsA_ref[...]) + mm(zB, wsB_ref[...]) +
                    hb_ref[...])

  return pl.pallas_call(
      kern,
      out_shape=jax.ShapeDtypeStruct((_NP, 128), jnp.float32),
  )(h_p, parts_p, E_bd, T_mat, *mats, *vecs)


def _finalize(parts_d):
  def kern(p_ref, out_ref):
    sacc = p_ref[0] + p_ref[1]
    sacc = sacc[0:N_GYM, :]
    cnt = sacc[:, 6:7]
    out_ref[...] = jnp.where(cnt > 0, sacc[:, :6] / jnp.maximum(cnt, 1.0),
                             0.0)

  return pl.pallas_call(
      kern,
      out_shape=jax.ShapeDtypeStruct((N_GYM, D), jnp.float32),
  )(parts_d)


def _pad_edges(src, dst, trash, nch, g):
  e = src.shape[0]
  tot = (NW * nch + 1) * g * LN
  src_p = jnp.concatenate(
      [src, jnp.zeros((tot - e,), jnp.int32)]).reshape(NW * nch + 1, g, LN)
  dst_p = jnp.concatenate(
      [dst, jnp.full((tot - e,), trash, jnp.int32)]).reshape(NW * nch + 1,
                                                             g, LN)
  return src_p, dst_p


def kernel(h, link_edge_index, direct_src, direct_dst, n_gym,
           fc1_W, fc1_b, fc2_W, fc2_b, sc1_W, sc1_b, bn1_w, bn1_b,
           fc3_W, fc3_b, fc4_W, fc4_b, sc2_W, sc2_b, bn2_w, bn2_b,
           fc5_W, fc5_b, fc6_W, fc6_b, sc3_W, sc3_b, bn3_w, bn3_b):
  del n_gym
  col = jnp.zeros((1, P), jnp.float32).at[0, 6].set(1.0)
  h0_p = (jnp.pad(h, ((0, 0), (0, 2))) + col).reshape(_NP, 128)

  z_main = jnp.zeros((_ACC_ROWS // NS, P), jnp.float32)
  dsrc_p, ddst_p = _pad_edges(direct_src, direct_dst, N_GYM,
                              _DIR_NCH, _DIR_G)
  z_dir = jnp.zeros((_DACC_ROWS // NS, P), jnp.float32)

  E_bd = _pick6()
  T_mat = _tilemat()
  layer_consts = [
      _dense_consts(fc1_W, fc1_b, bn1_w, bn1_b, fc2_W, fc2_b, sc1_W, sc1_b),
      _dense_consts(fc3_W, fc3_b, bn2_w, bn2_b, fc4_W, fc4_b, sc2_W, sc2_b),
      _dense_consts(fc5_W, fc5_b, bn3_w, bn3_b, fc6_W, fc6_b, sc3_W, sc3_b),
  ]

  hp = h0_p
  for mats, vecs in layer_consts:
    tab = hp.reshape(N_ROAD, P)
    parts = _agg_main(tab, link_edge_index, z_main)
    parts_p = parts.reshape(NC, _PR, 128)
    hp = _dense_layer(hp, parts_p, E_bd, T_mat, mats, vecs)

  parts_d = _agg_dir(hp.reshape(N_ROAD, P), dsrc_p, ddst_p, z_dir)
  return _finalize(parts_d)
